# Initial kernel scaffold; baseline (speedup 1.0000x reference)
import jax
import jax.numpy as jnp
from jax.experimental import pallas as pl


def kernel(pred_logits, pred_boxes, tgt_labels, tgt_boxes_xyxy, image_size_xyxy, image_size_xyxy_tgt):
    raise NotImplementedError("write your pallas kernel here")


# speedup vs baseline: 13.7142x; 13.7142x over previous
import functools

import jax
import jax.numpy as jnp
from jax import lax
from jax.experimental import pallas as pl
from jax.experimental.pallas import tpu as pltpu

_BS, _NQ, _NC, _NT = 16, 500, 80, 100
_ALPHA, _GAMMA = 0.25, 2.0
_CLASS_W, _L1_W, _GIOU_W = 2.0, 5.0, 2.0
_BIG = 1e30
_NQP = 512
_NTP = 128


def _focal_pn(logits):
    p = jax.nn.sigmoid(logits)
    neg = (1.0 - _ALPHA) * (p * p) * (-jnp.log(1.0 - p + 1e-08))
    pos = _ALPHA * ((1.0 - p) * (1.0 - p)) * (-jnp.log(p + 1e-08))
    return pos - neg


def _pair_costs(A, An, ids_row, tgtT, tgtnT, n_cls_lanes):
    pn = _focal_pn
    ax0, ay0 = An[:, 0:1], An[:, 1:2]
    ax1, ay1 = An[:, 2:3], An[:, 3:4]
    tx0n, ty0n = tgtnT[0:1, :], tgtnT[1:2, :]
    tx1n, ty1n = tgtnT[2:3, :], tgtnT[3:4, :]
    cost_bbox = (jnp.abs(ax0 - tx0n) + jnp.abs(ay0 - ty0n)
                 + jnp.abs(ax1 - tx1n) + jnp.abs(ay1 - ty1n))

    bx0, by0 = A[:, 0:1], A[:, 1:2]
    bx1, by1 = A[:, 2:3], A[:, 3:4]
    tx0, ty0 = tgtT[0:1, :], tgtT[1:2, :]
    tx1, ty1 = tgtT[2:3, :], tgtT[3:4, :]
    area1 = (bx1 - bx0) * (by1 - by0)
    area2 = (tx1 - tx0) * (ty1 - ty0)
    ltx = jnp.maximum(bx0, tx0)
    lty = jnp.maximum(by0, ty0)
    rbx = jnp.minimum(bx1, tx1)
    rby = jnp.minimum(by1, ty1)
    wx = jnp.maximum(rbx - ltx, 0.0)
    wy = jnp.maximum(rby - lty, 0.0)
    inter = wx * wy
    union = area1 + area2 - inter
    iou = inter / union
    ltx2 = jnp.minimum(bx0, tx0)
    lty2 = jnp.minimum(by0, ty0)
    rbx2 = jnp.maximum(bx1, tx1)
    rby2 = jnp.maximum(by1, ty1)
    wx2 = jnp.maximum(rbx2 - ltx2, 0.0)
    wy2 = jnp.maximum(rby2 - lty2, 0.0)
    area = wx2 * wy2
    giou = iou - (area - union) / area
    cost_giou = -giou
    return cost_bbox, cost_giou


def _cost_kernel(logits_ref, boxes_ref, isrow_ref, ids_ref, tgtT_ref,
                 tgtisT_ref, c_ref):
    R = logits_ref.shape[0]
    T = ids_ref.shape[1]
    pn = _focal_pn(logits_ref[...])
    ids = ids_ref[0:1, :]
    cls_iota = lax.broadcasted_iota(jnp.int32, (_NC, T), 0)
    oh = (cls_iota == ids).astype(jnp.float32)
    cost_class = lax.dot_general(
        pn, oh, (((1,), (0,)), ((), ())),
        precision=lax.Precision.HIGHEST,
        preferred_element_type=jnp.float32)
    A = boxes_ref[...]
    An = A / isrow_ref[...]
    tgtT = tgtT_ref[...]
    tgtnT = tgtT / tgtisT_ref[...]
    cost_bbox, cost_giou = _pair_costs(A, An, ids, tgtT, tgtnT, T)
    c_ref[...] = _L1_W * cost_bbox + _CLASS_W * cost_class + _GIOU_W * cost_giou


def _diag_kernel(logits_ref, boxes_ref, isrow_ref, ids_ref, tgtT_ref,
                 tgtisT_ref, cd_ref):
    pn = _focal_pn(logits_ref[...])
    ids = ids_ref[0, 0:1, :]
    cls_iota = lax.broadcasted_iota(jnp.int32, (_NC, _NTP), 0)
    oh = (cls_iota == ids).astype(jnp.float32)
    cost_class = lax.dot_general(
        pn, oh, (((1,), (0,)), ((), ())),
        precision=lax.Precision.HIGHEST,
        preferred_element_type=jnp.float32)
    A = boxes_ref[...]
    An = A / isrow_ref[...]
    tgtT = tgtT_ref[0]
    tgtnT = tgtT / tgtisT_ref[0]
    cost_bbox, cost_giou = _pair_costs(A, An, ids, tgtT, tgtnT, _NTP)
    cd = _L1_W * cost_bbox + _CLASS_W * cost_class + _GIOU_W * cost_giou
    rowi = lax.broadcasted_iota(jnp.int32, (_NQP, _NTP), 0)
    lanei = lax.broadcasted_iota(jnp.int32, (_NQP, _NTP), 1)
    cd_ref[0] = jnp.where((rowi >= _NQ) | (lanei >= _NT), _BIG, cd)


def _greedy_kernel(cd_ref, rows_ref, cols_ref, cur_ref):
    cur_ref[...] = cd_ref[0]
    lanei = lax.broadcasted_iota(jnp.int32, (1, _NTP), 1)
    flat = (lax.broadcasted_iota(jnp.int32, (_NQP, _NTP), 0) * _NTP
            + lax.broadcasted_iota(jnp.int32, (_NQP, _NTP), 1))

    def body(t, state):
        rowsv, colsv, colmask = state
        vals = cur_ref[...] + colmask
        m = jnp.min(vals)
        idx = jnp.min(jnp.where(vals == m, flat, jnp.int32(2 ** 30)))
        i = idx // _NTP
        j = idx - i * _NTP
        cur_ref[pl.ds(i, 1), :] = jnp.full((1, _NTP), _BIG, jnp.float32)
        colmask = jnp.where(lanei == j, _BIG, colmask)
        rowsv = jnp.where(lanei == t, i, rowsv)
        colsv = jnp.where(lanei == t, j, colsv)
        return rowsv, colsv, colmask

    z = jnp.zeros((1, _NTP), jnp.int32)
    rowsv, colsv, _ = lax.fori_loop(
        0, _NT, body, (z, z, jnp.zeros((1, _NTP), jnp.float32)))
    rows_ref[...] = rowsv.reshape(1, 1, _NTP)
    cols_ref[...] = colsv.reshape(1, 1, _NTP)


def kernel(pred_logits, pred_boxes, tgt_labels, tgt_boxes_xyxy,
           image_size_xyxy, image_size_xyxy_tgt):
    bs, nq, nc = pred_logits.shape
    nt = tgt_labels.shape[1]

    logits2d = pred_logits.reshape(bs * nq, nc)
    boxes2d = pred_boxes.reshape(bs * nq, 4)
    boxesP = jnp.pad(boxes2d, ((0, 0), (0, 124)))
    isrow = jnp.repeat(image_size_xyxy, nq, axis=0)
    isrowP = jnp.pad(isrow, ((0, 0), (0, 124)), constant_values=1.0)
    ids = tgt_labels.reshape(-1).astype(jnp.int32)
    idsRow = jnp.broadcast_to(ids[None, :], (8, bs * nt))
    tgtT = jnp.pad(tgt_boxes_xyxy.reshape(-1, 4).T, ((0, 4), (0, 0)))
    tgtisT = jnp.pad(image_size_xyxy_tgt.reshape(-1, 4).T, ((0, 4), (0, 0)),
                     constant_values=1.0)

    RB = 1000
    grid = (bs * nq) // RB
    T = bs * nt
    C2 = pl.pallas_call(
        _cost_kernel,
        grid=(grid,),
        in_specs=[
            pl.BlockSpec((RB, nc), lambda g: (g, 0)),
            pl.BlockSpec((RB, 128), lambda g: (g, 0)),
            pl.BlockSpec((RB, 128), lambda g: (g, 0)),
            pl.BlockSpec((8, T), lambda g: (0, 0)),
            pl.BlockSpec((8, T), lambda g: (0, 0)),
            pl.BlockSpec((8, T), lambda g: (0, 0)),
        ],
        out_specs=pl.BlockSpec((RB, T), lambda g: (g, 0)),
        out_shape=jax.ShapeDtypeStruct((bs * nq, T), jnp.float32),
    )(logits2d, boxesP, isrowP, idsRow, tgtT, tgtisT)
    C = C2.reshape(bs, nq, T)

    pad_q = _NQP - nq
    logitsPd = jnp.pad(pred_logits, ((0, 0), (0, pad_q), (0, 0))
                       ).reshape(bs * _NQP, nc)
    boxesPd = jnp.pad(pred_boxes, ((0, 0), (0, pad_q), (0, 0)))
    boxesPd = jnp.pad(boxesPd.reshape(bs * _NQP, 4), ((0, 0), (0, 124)))
    isrowPd = jnp.repeat(image_size_xyxy, _NQP, axis=0)
    isrowPd = jnp.pad(isrowPd, ((0, 0), (0, 124)), constant_values=1.0)
    idsD = jnp.pad(tgt_labels.astype(jnp.int32), ((0, 0), (0, _NTP - nt)))
    idsD = jnp.broadcast_to(idsD[:, None, :], (bs, 8, _NTP))
    tgtTd = jnp.pad(jnp.swapaxes(tgt_boxes_xyxy, 1, 2),
                    ((0, 0), (0, 4), (0, _NTP - nt)))
    tgtisTd = jnp.pad(jnp.swapaxes(image_size_xyxy_tgt, 1, 2),
                      ((0, 0), (0, 4), (0, _NTP - nt)), constant_values=1.0)

    Cd = pl.pallas_call(
        _diag_kernel,
        grid=(bs,),
        in_specs=[
            pl.BlockSpec((_NQP, nc), lambda b: (b, 0)),
            pl.BlockSpec((_NQP, 128), lambda b: (b, 0)),
            pl.BlockSpec((_NQP, 128), lambda b: (b, 0)),
            pl.BlockSpec((1, 8, _NTP), lambda b: (b, 0, 0)),
            pl.BlockSpec((1, 8, _NTP), lambda b: (b, 0, 0)),
            pl.BlockSpec((1, 8, _NTP), lambda b: (b, 0, 0)),
        ],
        out_specs=pl.BlockSpec((1, _NQP, _NTP), lambda b: (b, 0, 0)),
        out_shape=jax.ShapeDtypeStruct((bs, _NQP, _NTP), jnp.float32),
    )(logitsPd, boxesPd, isrowPd, idsD, tgtTd, tgtisTd)

    rows_out, cols_out = pl.pallas_call(
        _greedy_kernel,
        grid=(bs,),
        in_specs=[pl.BlockSpec((1, _NQP, _NTP), lambda b: (b, 0, 0))],
        out_specs=[pl.BlockSpec((1, 1, _NTP), lambda b: (b, 0, 0)),
                   pl.BlockSpec((1, 1, _NTP), lambda b: (b, 0, 0))],
        out_shape=[jax.ShapeDtypeStruct((bs, 1, _NTP), jnp.int32),
                   jax.ShapeDtypeStruct((bs, 1, _NTP), jnp.int32)],
        scratch_shapes=[pltpu.VMEM((_NQP, _NTP), jnp.float32)],
    )(Cd)

    rows = rows_out[:, 0, :nt]
    cols = cols_out[:, 0, :nt]
    return C, rows, cols
